# baseline (speedup 1.0000x reference)
import functools

import jax
import jax.numpy as jnp
from jax import lax
from jax.experimental import pallas as pl
from jax.experimental.pallas import tpu as pltpu

_N = 256
_K = 128
_D = 96


def _sortable_key(sim):
    i = lax.bitcast_convert_type(sim, jnp.int32)
    return jnp.where(i >= 0, i, i ^ jnp.int32(0x7FFFFFFF))


def _bitonic_argsort_cols(sim):
    n = sim.shape[0]
    key = _sortable_key(sim)
    idx = jax.lax.broadcasted_iota(jnp.int32, sim.shape, 0).astype(jnp.int16)
    pos = jax.lax.broadcasted_iota(jnp.int32, (n, 1), 0)

    size = 2
    while size <= n:
        d = size // 2
        dir_up = (pos & size) == 0
        while d >= 1:
            upper = (pos & d) != 0
            k_m = jnp.roll(key, -d, axis=0)
            k_p = jnp.roll(key, d, axis=0)
            i_m = jnp.roll(idx, -d, axis=0)
            i_p = jnp.roll(idx, d, axis=0)
            k_part = jnp.where(upper, k_p, k_m)
            i_part = jnp.where(upper, i_p, i_m)
            before = (key > k_part) | ((key == k_part) & (idx < i_part))
            keep = before ^ upper ^ (~dir_up)
            key = jnp.where(keep, key, k_part)
            idx = jnp.where(keep, idx, i_part)
            d //= 2
        size *= 2
    return idx


def _body(x_ref, out_ref):
    x = x_ref[0]
    dots = lax.dot_general(x, x, (((1,), (1,)), ((), ())),
                           precision=lax.Precision.DEFAULT,
                           preferred_element_type=jnp.float32)
    n2 = jnp.sum(x * x, axis=1, keepdims=True)
    norms = jnp.sqrt(n2)
    outer = norms * jnp.transpose(norms)
    sim = dots / jnp.maximum(outer, jnp.float32(1e-8))

    idx_sorted = _bitonic_argsort_cols(sim)
    idx_top = idx_sorted[:_K, :]

    idx16 = idx_top.astype(jnp.int16)
    iota_j = jax.lax.broadcasted_iota(jnp.int32, (_K, _N), 1).astype(jnp.int16)
    one = jnp.bfloat16(1.0)
    zero = jnp.bfloat16(0.0)
    c16 = jnp.zeros((_K, _N), jnp.bfloat16)
    for i0 in range(_N):
        col = idx16[:, i0:i0 + 1]
        c16 = c16 + jnp.where(col == iota_j, one, zero)
    c = c16.astype(jnp.float32)

    out = lax.dot_general(c, x, (((1,), (0,)), ((), ())),
                          preferred_element_type=jnp.float32)
    out_ref[0, 0] = out * jnp.float32(1.0 / _N)


@jax.jit
def kernel(clip_embeddings):
    b, n, d = clip_embeddings.shape
    return pl.pallas_call(
        _body,
        grid=(b,),
        in_specs=[pl.BlockSpec((1, n, d), lambda i: (i, 0, 0))],
        out_specs=pl.BlockSpec((1, 1, _K, d), lambda i: (i, 0, 0, 0)),
        out_shape=jax.ShapeDtypeStruct((b, 1, _K, d), jnp.float32),
        compiler_params=pltpu.CompilerParams(
            dimension_semantics=("arbitrary",),
        ),
    )(clip_embeddings)


# speedup vs baseline: 1.1235x; 1.1235x over previous
import functools

import jax
import jax.numpy as jnp
from jax import lax
from jax.experimental import pallas as pl
from jax.experimental.pallas import tpu as pltpu

_N = 256
_K = 128
_D = 96


def _sortable_key(sim):
    i = lax.bitcast_convert_type(sim, jnp.int32)
    return jnp.where(i >= 0, i, i ^ jnp.int32(0x7FFFFFFF))


def _bitonic_argsort_cols(sim):
    n = sim.shape[0]
    key = _sortable_key(sim)
    idx = jax.lax.broadcasted_iota(jnp.int32, sim.shape, 0)
    pos = jax.lax.broadcasted_iota(jnp.int32, (n, 1), 0)

    size = 2
    while size <= n:
        d = size // 2
        dir_up = (pos & size) == 0
        while d >= 1:
            upper = (pos & d) != 0
            k_m = jnp.roll(key, -d, axis=0)
            k_p = jnp.roll(key, d, axis=0)
            i_m = jnp.roll(idx, -d, axis=0)
            i_p = jnp.roll(idx, d, axis=0)
            k_part = jnp.where(upper, k_p, k_m)
            i_part = jnp.where(upper, i_p, i_m)
            before = (key > k_part) | ((key == k_part) & (idx < i_part))
            keep = before ^ upper ^ (~dir_up)
            key = jnp.where(keep, key, k_part)
            idx = jnp.where(keep, idx, i_part)
            d //= 2
        size *= 2
    return idx


def _body(x_ref, out_ref):
    x = x_ref[0]
    dots = lax.dot_general(x, x, (((1,), (1,)), ((), ())),
                           precision=lax.Precision.DEFAULT,
                           preferred_element_type=jnp.float32)
    n2 = jnp.sum(x * x, axis=1, keepdims=True)
    norms = jnp.sqrt(n2)
    outer = norms * jnp.transpose(norms)
    sim = dots / jnp.maximum(outer, jnp.float32(1e-8))

    strip = 128
    iota_j = jax.lax.broadcasted_iota(jnp.int32, (_K, _N), 1).astype(jnp.int16)
    one = jnp.bfloat16(1.0)
    zero = jnp.bfloat16(0.0)
    c16 = jnp.zeros((_K, _N), jnp.bfloat16)
    for s0 in range(0, _N, strip):
        idx_sorted = _bitonic_argsort_cols(sim[:, s0:s0 + strip])
        idx16 = idx_sorted[:_K, :].astype(jnp.int16)
        for i0 in range(strip):
            col = idx16[:, i0:i0 + 1]
            c16 = c16 + jnp.where(col == iota_j, one, zero)
    c = c16.astype(jnp.float32)

    out = lax.dot_general(c, x, (((1,), (0,)), ((), ())),
                          preferred_element_type=jnp.float32)
    out_ref[0, 0] = out * jnp.float32(1.0 / _N)


@jax.jit
def kernel(clip_embeddings):
    b, n, d = clip_embeddings.shape
    return pl.pallas_call(
        _body,
        grid=(b,),
        in_specs=[pl.BlockSpec((1, n, d), lambda i: (i, 0, 0))],
        out_specs=pl.BlockSpec((1, 1, _K, d), lambda i: (i, 0, 0, 0)),
        out_shape=jax.ShapeDtypeStruct((b, 1, _K, d), jnp.float32),
        compiler_params=pltpu.CompilerParams(
            dimension_semantics=("arbitrary",),
        ),
    )(clip_embeddings)
